# baseline (speedup 1.0000x reference)
import functools

import jax
import jax.numpy as jnp
from jax import lax
from jax.experimental import pallas as pl
from jax.experimental.pallas import tpu as pltpu
from jax.experimental.pallas import tpu_sc as plsc

N = 10000
NP = 10240
E = 320000
D = 128
B = 128
NTILES = 32
NBLK = 80
NBLK_TOT = NTILES * NBLK
EPAD = NBLK_TOT * B
STRIPE = NP // 16
NBUF = 4

_MESH = plsc.VectorSubcoreMesh(core_axis_name="c", subcore_axis_name="s")


def _deg_body(dst_hbm, out_hbm, dst_v, ones_v, zero_v, acc_sh):
    c = lax.axis_index("c")
    s = lax.axis_index("s")
    wid = c * 16 + s
    pltpu.sync_copy(dst_hbm.at[pl.ds(wid * NBLK, NBLK)], dst_v)

    def fill(i, carry):
        ones_v[i, :] = jnp.ones((16,), jnp.float32)
        zero_v[i, :] = jnp.zeros((16,), jnp.float32)
        return carry

    lax.fori_loop(0, B, fill, 0)

    base = s * STRIPE
    for t in range(5):
        pltpu.sync_copy(zero_v, acc_sh.at[pl.ds(base + t * B, B)])
    plsc.subcore_barrier()

    def blk(j, carry):
        pltpu.sync_copy(ones_v, acc_sh.at[dst_v.at[j]], add=True)
        return carry

    lax.fori_loop(0, NBLK, blk, 0)
    plsc.subcore_barrier()
    pltpu.sync_copy(acc_sh.at[pl.ds(base, STRIPE)],
                    out_hbm.at[c, pl.ds(base, STRIPE)])


@functools.partial(
    pl.kernel,
    out_type=pltpu.HBM((2, NP, 16), jnp.float32),
    mesh=_MESH,
    scratch_types=[
        pltpu.VMEM((NBLK, B), jnp.int32),
        pltpu.VMEM((B, 16), jnp.float32),
        pltpu.VMEM((B, 16), jnp.float32),
        pltpu.VMEM_SHARED((NP, 16), jnp.float32),
    ],
)
def _deg_kernel(dst_hbm, out_hbm, dst_v, ones_v, zero_v, acc_sh):
    _deg_body(dst_hbm, out_hbm, dst_v, ones_v, zero_v, acc_sh)


NCHUNK = 2
CBLK = NBLK // NCHUNK


def _scat_body(xs_hbm, src_hbm, dst_hbm, out_hbm,
               src_v, dst_v, buf0, buf1, acc_sh, sem0, sem1, sem2, sem3):
    c = lax.axis_index("c")
    s = lax.axis_index("s")
    wid = c * 16 + s

    bufs = [buf0, buf1]
    sems = [sem0, sem1, sem2, sem3]

    def z(i, carry):
        for k in range(D // 16):
            buf0[i, pl.ds(k * 16, 16)] = jnp.zeros((16,), jnp.float32)
        return carry

    lax.fori_loop(0, B, z, 0)
    base = s * STRIPE
    for t in range(5):
        pltpu.sync_copy(buf0, acc_sh.at[pl.ds(base + t * B, B)])
    plsc.subcore_barrier()

    H = B // 2

    def gather(i, b):
        pltpu.async_copy(xs_hbm.at[src_v.at[i, pl.ds(0, H)]],
                         bufs[b].at[pl.ds(0, H)], sems[2 * b])
        pltpu.async_copy(xs_hbm.at[src_v.at[i, pl.ds(H, H)]],
                         bufs[b].at[pl.ds(H, H)], sems[2 * b + 1])

    def gwait(i, b):
        pltpu.make_async_copy(xs_hbm.at[src_v.at[i, pl.ds(0, H)]],
                              bufs[b].at[pl.ds(0, H)], sems[2 * b]).wait()
        pltpu.make_async_copy(xs_hbm.at[src_v.at[i, pl.ds(H, H)]],
                              bufs[b].at[pl.ds(H, H)], sems[2 * b + 1]).wait()

    for ci in range(NCHUNK):
        cbase = wid * NBLK + ci * CBLK
        pltpu.sync_copy(src_hbm.at[pl.ds(cbase, CBLK)], src_v)
        pltpu.sync_copy(dst_hbm.at[pl.ds(cbase, CBLK)], dst_v)
        for b in range(2):
            gather(b, b)

        def ring(j, carry):
            for b in range(2):
                i = j * 2 + b
                gwait(i, b)
                pltpu.sync_copy(bufs[b], acc_sh.at[dst_v.at[i]], add=True)
                gather(i + 2, b)
            return carry

        lax.fori_loop(0, (CBLK - 2) // 2, ring, 0)

        for b in range(2):
            i = CBLK - 2 + b
            gwait(i, b)
            pltpu.sync_copy(bufs[b], acc_sh.at[dst_v.at[i]], add=True)

    plsc.subcore_barrier()
    pltpu.sync_copy(acc_sh.at[pl.ds(base, STRIPE)],
                    out_hbm.at[c, pl.ds(base, STRIPE)])


@functools.partial(
    pl.kernel,
    out_type=pltpu.HBM((2, NP, D), jnp.float32),
    mesh=_MESH,
    scratch_types=[
        pltpu.VMEM((CBLK, B), jnp.int32),
        pltpu.VMEM((CBLK, B), jnp.int32),
        pltpu.VMEM((B, D), jnp.float32),
        pltpu.VMEM((B, D), jnp.float32),
        pltpu.VMEM_SHARED((NP, D), jnp.float32),
        pltpu.SemaphoreType.DMA,
        pltpu.SemaphoreType.DMA,
        pltpu.SemaphoreType.DMA,
        pltpu.SemaphoreType.DMA,
    ],
)
def _scat_kernel(xs_hbm, src_hbm, dst_hbm, out_hbm,
                 src_v, dst_v, buf0, buf1, acc_sh, sem0, sem1, sem2, sem3):
    _scat_body(xs_hbm, src_hbm, dst_hbm, out_hbm,
               src_v, dst_v, buf0, buf1, acc_sh, sem0, sem1, sem2, sem3)


def _scale_body(degp_ref, x_ref, xs_ref):
    deg = degp_ref[0, :, 0:1] + degp_ref[1, :, 0:1] + 1.0
    dinv = lax.rsqrt(deg)
    xs_ref[...] = x_ref[...] * dinv


def _scale_call(degp, x_pad):
    return pl.pallas_call(
        _scale_body,
        out_shape=jax.ShapeDtypeStruct((NP, D), jnp.float32),
    )(degp, x_pad)


def _out_body(acc_ref, degp_ref, x_ref, wmu_ref, bmu_ref, wvar_ref, bvar_ref,
              mu_ref, var_ref):
    deg = degp_ref[0, 0:N, 0:1] + degp_ref[1, 0:N, 0:1] + 1.0
    dinv = lax.rsqrt(deg)
    ssum = acc_ref[0, 0:N, :] + acc_ref[1, 0:N, :]
    agg = dinv * (ssum + dinv * x_ref[...])
    mu_ref[...] = jnp.dot(agg, wmu_ref[...],
                          preferred_element_type=jnp.float32) + bmu_ref[...]
    var_ref[...] = jnp.dot(agg, wvar_ref[...],
                           preferred_element_type=jnp.float32) + bvar_ref[...]


def _out_call(acc, degp, x, Wmu, bmu, Wvar, bvar):
    return pl.pallas_call(
        _out_body,
        out_shape=(
            jax.ShapeDtypeStruct((N, D), jnp.float32),
            jax.ShapeDtypeStruct((N, D), jnp.float32),
        ),
    )(acc, degp, x, Wmu, bmu, Wvar, bvar)


def kernel(x, edge_index, Wmu, bmu, Wvar, bvar):
    src = edge_index[0]
    dst = edge_index[1]
    pad = jnp.full((EPAD - E,), N, dtype=jnp.int32)
    srcp = jnp.concatenate([src, pad]).reshape(NBLK_TOT, B)
    dstp = jnp.concatenate([dst, pad]).reshape(NBLK_TOT, B)
    x_pad = jnp.concatenate([x, jnp.zeros((NP - N, D), x.dtype)], axis=0)

    degp = _deg_kernel(dstp)
    xs = _scale_call(degp, x_pad)
    acc = _scat_kernel(xs, srcp, dstp)
    mu, var = _out_call(acc, degp, x, Wmu, bmu.reshape(1, D),
                        Wvar, bvar.reshape(1, D))
    return (mu, var)


# speedup vs baseline: 2.5175x; 2.5175x over previous
import functools

import jax
import jax.numpy as jnp
from jax import lax
from jax.experimental import pallas as pl
from jax.experimental.pallas import tpu as pltpu
from jax.experimental.pallas import tpu_sc as plsc

N = 10000
NP = 10240
E = 320000
D = 128
B = 128
NTILES = 32
NBLK = 80
NBLK_TOT = NTILES * NBLK
EPAD = NBLK_TOT * B
STRIPE = NP // 16
NBUF = 4

_MESH = plsc.VectorSubcoreMesh(core_axis_name="c", subcore_axis_name="s")


def _deg_body(dst_hbm, out_hbm, dst_v, ones_v, zero_v, acc_sh):
    c = lax.axis_index("c")
    s = lax.axis_index("s")
    wid = c * 16 + s
    pltpu.sync_copy(dst_hbm.at[pl.ds(wid * NBLK, NBLK)], dst_v)

    def fill(i, carry):
        ones_v[i, :] = jnp.ones((16,), jnp.float32)
        zero_v[i, :] = jnp.zeros((16,), jnp.float32)
        return carry

    lax.fori_loop(0, B, fill, 0)

    base = s * STRIPE
    for t in range(5):
        pltpu.sync_copy(zero_v, acc_sh.at[pl.ds(base + t * B, B)])
    plsc.subcore_barrier()

    def blk(j, carry):
        pltpu.sync_copy(ones_v, acc_sh.at[dst_v.at[j]], add=True)
        return carry

    lax.fori_loop(0, NBLK, blk, 0)
    plsc.subcore_barrier()
    pltpu.sync_copy(acc_sh.at[pl.ds(base, STRIPE)],
                    out_hbm.at[c, pl.ds(base, STRIPE)])


@functools.partial(
    pl.kernel,
    out_type=pltpu.HBM((2, NP, 16), jnp.float32),
    mesh=_MESH,
    scratch_types=[
        pltpu.VMEM((NBLK, B), jnp.int32),
        pltpu.VMEM((B, 16), jnp.float32),
        pltpu.VMEM((B, 16), jnp.float32),
        pltpu.VMEM_SHARED((NP, 16), jnp.float32),
    ],
)
def _deg_kernel(dst_hbm, out_hbm, dst_v, ones_v, zero_v, acc_sh):
    _deg_body(dst_hbm, out_hbm, dst_v, ones_v, zero_v, acc_sh)


NCHUNK = 2
CBLK = NBLK // NCHUNK


def _scat_body(xs_hbm, src_hbm, dst_hbm, out_hbm,
               src_v, dst_v, buf0, buf1, acc_sh, sem0, sem1, sem2, sem3):
    c = lax.axis_index("c")
    s = lax.axis_index("s")
    wid = c * 16 + s

    bufs = [buf0, buf1]
    sems = [sem0, sem1, sem2, sem3]

    def z(i, carry):
        for k in range(D // 16):
            buf0[i, pl.ds(k * 16, 16)] = jnp.zeros((16,), jnp.float32)
        return carry

    lax.fori_loop(0, B, z, 0)
    base = s * STRIPE
    for t in range(5):
        pltpu.sync_copy(buf0, acc_sh.at[pl.ds(base + t * B, B)])
    plsc.subcore_barrier()

    H = B // 2

    def gather(i, b):
        pltpu.async_copy(xs_hbm.at[src_v.at[i, pl.ds(0, H)]],
                         bufs[b].at[pl.ds(0, H)], sems[2 * b])
        pltpu.async_copy(xs_hbm.at[src_v.at[i, pl.ds(H, H)]],
                         bufs[b].at[pl.ds(H, H)], sems[2 * b + 1])

    def gwait(i, b):
        pltpu.make_async_copy(xs_hbm.at[src_v.at[i, pl.ds(0, H)]],
                              bufs[b].at[pl.ds(0, H)], sems[2 * b]).wait()
        pltpu.make_async_copy(xs_hbm.at[src_v.at[i, pl.ds(H, H)]],
                              bufs[b].at[pl.ds(H, H)], sems[2 * b + 1]).wait()

    for ci in range(NCHUNK):
        cbase = wid * NBLK + ci * CBLK
        pltpu.sync_copy(src_hbm.at[pl.ds(cbase, CBLK)], src_v)
        pltpu.sync_copy(dst_hbm.at[pl.ds(cbase, CBLK)], dst_v)
        for b in range(2):
            gather(b, b)

        def ring(j, carry):
            for b in range(2):
                i = j * 2 + b
                gwait(i, b)
                pltpu.sync_copy(bufs[b], acc_sh.at[dst_v.at[i]], add=True)
                gather(i + 2, b)
            return carry

        lax.fori_loop(0, (CBLK - 2) // 2, ring, 0)

        for b in range(2):
            i = CBLK - 2 + b
            gwait(i, b)
            pltpu.sync_copy(bufs[b], acc_sh.at[dst_v.at[i]], add=True)

    plsc.subcore_barrier()
    pltpu.sync_copy(acc_sh.at[pl.ds(base, STRIPE)],
                    out_hbm.at[c, pl.ds(base, STRIPE)])


@functools.partial(
    pl.kernel,
    out_type=pltpu.HBM((2, NP, D), jnp.float32),
    mesh=_MESH,
    scratch_types=[
        pltpu.VMEM((CBLK, B), jnp.int32),
        pltpu.VMEM((CBLK, B), jnp.int32),
        pltpu.VMEM((B, D), jnp.float32),
        pltpu.VMEM((B, D), jnp.float32),
        pltpu.VMEM_SHARED((NP, D), jnp.float32),
        pltpu.SemaphoreType.DMA,
        pltpu.SemaphoreType.DMA,
        pltpu.SemaphoreType.DMA,
        pltpu.SemaphoreType.DMA,
    ],
)
def _scat_kernel(xs_hbm, src_hbm, dst_hbm, out_hbm,
                 src_v, dst_v, buf0, buf1, acc_sh, sem0, sem1, sem2, sem3):
    _scat_body(xs_hbm, src_hbm, dst_hbm, out_hbm,
               src_v, dst_v, buf0, buf1, acc_sh, sem0, sem1, sem2, sem3)


def _scale_body(degp_ref, x_ref, xs_ref):
    deg = degp_ref[0, :, 0:1] + degp_ref[1, :, 0:1] + 1.0
    dinv = lax.rsqrt(deg)
    xs_ref[...] = x_ref[...] * dinv


def _scale_call(degp, x_pad):
    return pl.pallas_call(
        _scale_body,
        out_shape=jax.ShapeDtypeStruct((NP, D), jnp.float32),
    )(degp, x_pad)


def _out_body(acc_ref, degp_ref, x_ref, wmu_ref, bmu_ref, wvar_ref, bvar_ref,
              mu_ref, var_ref):
    deg = degp_ref[0, 0:N, 0:1] + degp_ref[1, 0:N, 0:1] + 1.0
    dinv = lax.rsqrt(deg)
    ssum = acc_ref[0, 0:N, :] + acc_ref[1, 0:N, :]
    agg = dinv * (ssum + dinv * x_ref[...])
    mu_ref[...] = jnp.dot(agg, wmu_ref[...],
                          preferred_element_type=jnp.float32) + bmu_ref[...]
    var_ref[...] = jnp.dot(agg, wvar_ref[...],
                           preferred_element_type=jnp.float32) + bvar_ref[...]


def _out_call(acc, degp, x, Wmu, bmu, Wvar, bvar):
    return pl.pallas_call(
        _out_body,
        out_shape=(
            jax.ShapeDtypeStruct((N, D), jnp.float32),
            jax.ShapeDtypeStruct((N, D), jnp.float32),
        ),
    )(acc, degp, x, Wmu, bmu, Wvar, bvar)


def kernel(x, edge_index, Wmu, bmu, Wvar, bvar):
    src = edge_index[0]
    dst = edge_index[1]
    pad_src = jnp.arange(EPAD - E, dtype=jnp.int32) % N
    pad_dst = N + jnp.arange(EPAD - E, dtype=jnp.int32) % (NP - N)
    srcp = jnp.concatenate([src, pad_src]).reshape(NBLK_TOT, B)
    dstp = jnp.concatenate([dst, pad_dst]).reshape(NBLK_TOT, B)
    x_pad = jnp.concatenate([x, jnp.zeros((NP - N, D), x.dtype)], axis=0)

    degp = _deg_kernel(dstp)
    xs = _scale_call(degp, x_pad)
    acc = _scat_kernel(xs, srcp, dstp)
    mu, var = _out_call(acc, degp, x, Wmu, bmu.reshape(1, D),
                        Wvar, bvar.reshape(1, D))
    return (mu, var)


# speedup vs baseline: 2.6751x; 1.0626x over previous
import functools

import jax
import jax.numpy as jnp
import numpy as np
from jax import lax
from jax.experimental import pallas as pl
from jax.experimental.pallas import tpu as pltpu
from jax.experimental.pallas import tpu_sc as plsc

N = 10000
NP = 10240
E = 320000
D = 128
B = 128
NTILES = 32
NBLK = 80
NBLK_TOT = NTILES * NBLK
EPAD = NBLK_TOT * B
STRIPE = NP // 16
NBUF = 4

_MESH = plsc.VectorSubcoreMesh(core_axis_name="c", subcore_axis_name="s")


def _deg_body(ei_hbm, out_hbm, dst_v, ones_v, zero_v, acc_sh):
    c = lax.axis_index("c")
    s = lax.axis_index("s")
    wid = c * 16 + s
    pltpu.sync_copy(ei_hbm.at[1, pl.ds(wid * NBLK, NBLK)], dst_v)

    def fill(i, carry):
        ones_v[i, :] = jnp.ones((16,), jnp.float32)
        zero_v[i, :] = jnp.zeros((16,), jnp.float32)
        return carry

    lax.fori_loop(0, B, fill, 0)

    base = s * STRIPE
    for t in range(5):
        pltpu.sync_copy(zero_v, acc_sh.at[pl.ds(base + t * B, B)])
    plsc.subcore_barrier()

    def blk(j, carry):
        pltpu.sync_copy(ones_v, acc_sh.at[dst_v.at[j]], add=True)
        return carry

    lax.fori_loop(0, NBLK, blk, 0)
    plsc.subcore_barrier()
    pltpu.sync_copy(acc_sh.at[pl.ds(base, STRIPE)],
                    out_hbm.at[c, pl.ds(base, STRIPE)])


@functools.partial(
    pl.kernel,
    out_type=pltpu.HBM((2, NP, 16), jnp.float32),
    mesh=_MESH,
    scratch_types=[
        pltpu.VMEM((NBLK, B), jnp.int32),
        pltpu.VMEM((B, 16), jnp.float32),
        pltpu.VMEM((B, 16), jnp.float32),
        pltpu.VMEM_SHARED((NP, 16), jnp.float32),
    ],
)
def _deg_kernel(ei_hbm, out_hbm, dst_v, ones_v, zero_v, acc_sh):
    _deg_body(ei_hbm, out_hbm, dst_v, ones_v, zero_v, acc_sh)


NCHUNK = 2
CBLK = NBLK // NCHUNK


def _scat_body(xs_hbm, ei_hbm, out_hbm,
               src_v, dst_v, buf0, buf1, acc_sh, sem0, sem1, sem2, sem3):
    c = lax.axis_index("c")
    s = lax.axis_index("s")
    wid = c * 16 + s

    bufs = [buf0, buf1]
    sems = [sem0, sem1, sem2, sem3]

    def z(i, carry):
        for k in range(D // 16):
            buf0[i, pl.ds(k * 16, 16)] = jnp.zeros((16,), jnp.float32)
        return carry

    lax.fori_loop(0, B, z, 0)
    base = s * STRIPE
    for t in range(5):
        pltpu.sync_copy(buf0, acc_sh.at[pl.ds(base + t * B, B)])
    plsc.subcore_barrier()

    H = B // 2

    def gather(i, b):
        pltpu.async_copy(xs_hbm.at[src_v.at[i, pl.ds(0, H)]],
                         bufs[b].at[pl.ds(0, H)], sems[2 * b])
        pltpu.async_copy(xs_hbm.at[src_v.at[i, pl.ds(H, H)]],
                         bufs[b].at[pl.ds(H, H)], sems[2 * b + 1])

    def gwait(i, b):
        pltpu.make_async_copy(xs_hbm.at[src_v.at[i, pl.ds(0, H)]],
                              bufs[b].at[pl.ds(0, H)], sems[2 * b]).wait()
        pltpu.make_async_copy(xs_hbm.at[src_v.at[i, pl.ds(H, H)]],
                              bufs[b].at[pl.ds(H, H)], sems[2 * b + 1]).wait()

    for ci in range(NCHUNK):
        cbase = wid * NBLK + ci * CBLK
        pltpu.sync_copy(ei_hbm.at[0, pl.ds(cbase, CBLK)], src_v)
        pltpu.sync_copy(ei_hbm.at[1, pl.ds(cbase, CBLK)], dst_v)
        for b in range(2):
            gather(b, b)

        def ring(j, carry):
            for b in range(2):
                i = j * 2 + b
                gwait(i, b)
                pltpu.sync_copy(bufs[b], acc_sh.at[dst_v.at[i]], add=True)
                gather(i + 2, b)
            return carry

        lax.fori_loop(0, (CBLK - 2) // 2, ring, 0)

        for b in range(2):
            i = CBLK - 2 + b
            gwait(i, b)
            pltpu.sync_copy(bufs[b], acc_sh.at[dst_v.at[i]], add=True)

    plsc.subcore_barrier()
    pltpu.sync_copy(acc_sh.at[pl.ds(base, STRIPE)],
                    out_hbm.at[c, pl.ds(base, STRIPE)])


@functools.partial(
    pl.kernel,
    out_type=pltpu.HBM((2, NP, D), jnp.float32),
    mesh=_MESH,
    scratch_types=[
        pltpu.VMEM((CBLK, B), jnp.int32),
        pltpu.VMEM((CBLK, B), jnp.int32),
        pltpu.VMEM((B, D), jnp.float32),
        pltpu.VMEM((B, D), jnp.float32),
        pltpu.VMEM_SHARED((NP, D), jnp.float32),
        pltpu.SemaphoreType.DMA,
        pltpu.SemaphoreType.DMA,
        pltpu.SemaphoreType.DMA,
        pltpu.SemaphoreType.DMA,
    ],
)
def _scat_kernel(xs_hbm, ei_hbm, out_hbm,
                 src_v, dst_v, buf0, buf1, acc_sh, sem0, sem1, sem2, sem3):
    _scat_body(xs_hbm, ei_hbm, out_hbm,
               src_v, dst_v, buf0, buf1, acc_sh, sem0, sem1, sem2, sem3)


def _scale_body(degp_ref, x_ref, xs_ref):
    deg = degp_ref[0, 0:N, 0:1] + degp_ref[1, 0:N, 0:1] + 1.0
    dinv = lax.rsqrt(deg)
    xs_ref[0:N, :] = x_ref[...] * dinv
    xs_ref[N:NP, :] = jnp.zeros((NP - N, D), jnp.float32)


def _scale_call(degp, x):
    return pl.pallas_call(
        _scale_body,
        out_shape=jax.ShapeDtypeStruct((NP, D), jnp.float32),
    )(degp, x)


def _out_body(acc_ref, degp_ref, x_ref, wmu_ref, bmu_ref, wvar_ref, bvar_ref,
              mu_ref, var_ref):
    deg = degp_ref[0, 0:N, 0:1] + degp_ref[1, 0:N, 0:1] + 1.0
    dinv = lax.rsqrt(deg)
    ssum = acc_ref[0, 0:N, :] + acc_ref[1, 0:N, :]
    agg = dinv * (ssum + dinv * x_ref[...])
    mu_ref[...] = jnp.dot(agg, wmu_ref[...],
                          preferred_element_type=jnp.float32) + bmu_ref[...]
    var_ref[...] = jnp.dot(agg, wvar_ref[...],
                           preferred_element_type=jnp.float32) + bvar_ref[...]


def _out_call(acc, degp, x, Wmu, bmu, Wvar, bvar):
    return pl.pallas_call(
        _out_body,
        out_shape=(
            jax.ShapeDtypeStruct((N, D), jnp.float32),
            jax.ShapeDtypeStruct((N, D), jnp.float32),
        ),
    )(acc, degp, x, Wmu, bmu, Wvar, bvar)


def kernel(x, edge_index, Wmu, bmu, Wvar, bvar):
    pad_blk = jnp.asarray(np.stack([
        np.arange(EPAD - E, dtype=np.int32) % N,
        N + np.arange(EPAD - E, dtype=np.int32) % (NP - N),
    ]))
    eip = jnp.concatenate([edge_index, pad_blk], axis=1)
    eip = eip.reshape(2, NBLK_TOT, B)

    degp = _deg_kernel(eip)
    xs = _scale_call(degp, x)
    acc = _scat_kernel(xs, eip)
    mu, var = _out_call(acc, degp, x, Wmu, bmu.reshape(1, D),
                        Wvar, bvar.reshape(1, D))
    return (mu, var)


# speedup vs baseline: 3.0007x; 1.1217x over previous
import functools

import jax
import jax.numpy as jnp
import numpy as np
from jax import lax
from jax.experimental import pallas as pl
from jax.experimental.pallas import tpu as pltpu
from jax.experimental.pallas import tpu_sc as plsc

N = 10000
NP = 10240
E = 320000
D = 128
B = 128
NTILES = 32
NBLK = 80
NBLK_TOT = NTILES * NBLK
EPAD = NBLK_TOT * B
STRIPE = NP // 16
NBUF = 4

_MESH = plsc.VectorSubcoreMesh(core_axis_name="c", subcore_axis_name="s")


def _deg_body(ei_hbm, out_hbm, dst_v, hist_v):
    c = lax.axis_index("c")
    s = lax.axis_index("s")
    wid = c * 16 + s
    pltpu.sync_copy(ei_hbm.at[1, pl.ds(wid * NBLK, NBLK)], dst_v)

    def zero(i, carry):
        hist_v[pl.ds(i * 16, 16)] = jnp.zeros((16,), jnp.float32)
        return carry

    lax.fori_loop(0, NP // 16, zero, 0)
    ones = jnp.ones((16,), jnp.float32)

    def blk(j, carry):
        for k in range(B // 16):
            idx = dst_v[j, pl.ds(k * 16, 16)]
            plsc.addupdate_scatter(hist_v, [idx], ones)
        return carry

    lax.fori_loop(0, NBLK, blk, 0)
    pltpu.sync_copy(hist_v, out_hbm.at[c, s])


@functools.partial(
    pl.kernel,
    out_type=pltpu.HBM((2, 16, NP), jnp.float32),
    mesh=_MESH,
    scratch_types=[
        pltpu.VMEM((NBLK, B), jnp.int32),
        pltpu.VMEM((NP,), jnp.float32),
    ],
    compiler_params=pltpu.CompilerParams(needs_layout_passes=False),
)
def _deg_kernel(ei_hbm, out_hbm, dst_v, hist_v):
    _deg_body(ei_hbm, out_hbm, dst_v, hist_v)


NCHUNK = 2
CBLK = NBLK // NCHUNK


def _scat_body(xs_hbm, ei_hbm, out_hbm,
               src_v, dst_v, buf0, buf1, acc_sh, sem0, sem1, sem2, sem3):
    c = lax.axis_index("c")
    s = lax.axis_index("s")
    wid = c * 16 + s

    bufs = [buf0, buf1]
    sems = [sem0, sem1, sem2, sem3]

    def z(i, carry):
        for k in range(D // 16):
            buf0[i, pl.ds(k * 16, 16)] = jnp.zeros((16,), jnp.float32)
        return carry

    lax.fori_loop(0, B, z, 0)
    base = s * STRIPE
    for t in range(5):
        pltpu.sync_copy(buf0, acc_sh.at[pl.ds(base + t * B, B)])
    plsc.subcore_barrier()

    H = B // 2

    def gather(i, b):
        pltpu.async_copy(xs_hbm.at[src_v.at[i, pl.ds(0, H)]],
                         bufs[b].at[pl.ds(0, H)], sems[2 * b])
        pltpu.async_copy(xs_hbm.at[src_v.at[i, pl.ds(H, H)]],
                         bufs[b].at[pl.ds(H, H)], sems[2 * b + 1])

    def gwait(i, b):
        pltpu.make_async_copy(xs_hbm.at[src_v.at[i, pl.ds(0, H)]],
                              bufs[b].at[pl.ds(0, H)], sems[2 * b]).wait()
        pltpu.make_async_copy(xs_hbm.at[src_v.at[i, pl.ds(H, H)]],
                              bufs[b].at[pl.ds(H, H)], sems[2 * b + 1]).wait()

    for ci in range(NCHUNK):
        cbase = wid * NBLK + ci * CBLK
        pltpu.sync_copy(ei_hbm.at[0, pl.ds(cbase, CBLK)], src_v)
        pltpu.sync_copy(ei_hbm.at[1, pl.ds(cbase, CBLK)], dst_v)
        for b in range(2):
            gather(b, b)

        def ring(j, carry):
            for b in range(2):
                i = j * 2 + b
                gwait(i, b)
                pltpu.sync_copy(bufs[b], acc_sh.at[dst_v.at[i]], add=True)
                gather(i + 2, b)
            return carry

        lax.fori_loop(0, (CBLK - 2) // 2, ring, 0)

        for b in range(2):
            i = CBLK - 2 + b
            gwait(i, b)
            pltpu.sync_copy(bufs[b], acc_sh.at[dst_v.at[i]], add=True)

    plsc.subcore_barrier()
    pltpu.sync_copy(acc_sh.at[pl.ds(base, STRIPE)],
                    out_hbm.at[c, pl.ds(base, STRIPE)])


@functools.partial(
    pl.kernel,
    out_type=pltpu.HBM((2, NP, D), jnp.float32),
    mesh=_MESH,
    scratch_types=[
        pltpu.VMEM((CBLK, B), jnp.int32),
        pltpu.VMEM((CBLK, B), jnp.int32),
        pltpu.VMEM((B, D), jnp.float32),
        pltpu.VMEM((B, D), jnp.float32),
        pltpu.VMEM_SHARED((NP, D), jnp.float32),
        pltpu.SemaphoreType.DMA,
        pltpu.SemaphoreType.DMA,
        pltpu.SemaphoreType.DMA,
        pltpu.SemaphoreType.DMA,
    ],
)
def _scat_kernel(xs_hbm, ei_hbm, out_hbm,
                 src_v, dst_v, buf0, buf1, acc_sh, sem0, sem1, sem2, sem3):
    _scat_body(xs_hbm, ei_hbm, out_hbm,
               src_v, dst_v, buf0, buf1, acc_sh, sem0, sem1, sem2, sem3)


def _scale_body(degp_ref, x_ref, xs_ref):
    deg = (jnp.sum(degp_ref[0, :, 0:N], axis=0)
           + jnp.sum(degp_ref[1, :, 0:N], axis=0) + 1.0)
    dinv = lax.rsqrt(deg)[:, None]
    xs_ref[0:N, :] = x_ref[...] * dinv
    xs_ref[N:NP, :] = jnp.zeros((NP - N, D), jnp.float32)


def _scale_call(degp, x):
    return pl.pallas_call(
        _scale_body,
        out_shape=jax.ShapeDtypeStruct((NP, D), jnp.float32),
    )(degp, x)


def _out_body(acc_ref, degp_ref, x_ref, wmu_ref, bmu_ref, wvar_ref, bvar_ref,
              mu_ref, var_ref):
    deg = (jnp.sum(degp_ref[0, :, 0:N], axis=0)
           + jnp.sum(degp_ref[1, :, 0:N], axis=0) + 1.0)
    dinv = lax.rsqrt(deg)[:, None]
    ssum = acc_ref[0, 0:N, :] + acc_ref[1, 0:N, :]
    agg = dinv * (ssum + dinv * x_ref[...])
    mu_ref[...] = jnp.dot(agg, wmu_ref[...],
                          preferred_element_type=jnp.float32) + bmu_ref[...]
    var_ref[...] = jnp.dot(agg, wvar_ref[...],
                           preferred_element_type=jnp.float32) + bvar_ref[...]


def _out_call(acc, degp, x, Wmu, bmu, Wvar, bvar):
    return pl.pallas_call(
        _out_body,
        out_shape=(
            jax.ShapeDtypeStruct((N, D), jnp.float32),
            jax.ShapeDtypeStruct((N, D), jnp.float32),
        ),
    )(acc, degp, x, Wmu, bmu, Wvar, bvar)


def kernel(x, edge_index, Wmu, bmu, Wvar, bvar):
    pad_blk = jnp.asarray(np.stack([
        np.arange(EPAD - E, dtype=np.int32) % N,
        N + np.arange(EPAD - E, dtype=np.int32) % (NP - N),
    ]))
    eip = jnp.concatenate([edge_index, pad_blk], axis=1)
    eip = eip.reshape(2, NBLK_TOT, B)

    degp = _deg_kernel(eip)
    xs = _scale_call(degp, x)
    acc = _scat_kernel(xs, eip)
    mu, var = _out_call(acc, degp, x, Wmu, bmu.reshape(1, D),
                        Wvar, bvar.reshape(1, D))
    return (mu, var)


# speedup vs baseline: 3.0410x; 1.0134x over previous
import functools

import jax
import jax.numpy as jnp
import numpy as np
from jax import lax
from jax.experimental import pallas as pl
from jax.experimental.pallas import tpu as pltpu
from jax.experimental.pallas import tpu_sc as plsc

N = 10000
NP = 10240
E = 320000
D = 128
B = 128
NTILES = 32
NBLK = 80
NBLK_TOT = NTILES * NBLK
EPAD = NBLK_TOT * B
STRIPE = NP // 16
NBUF = 4

_MESH = plsc.VectorSubcoreMesh(core_axis_name="c", subcore_axis_name="s")


def _deg_body(ei_hbm, out_hbm, dst_v, hist_v):
    c = lax.axis_index("c")
    s = lax.axis_index("s")
    wid = c * 16 + s
    pltpu.sync_copy(ei_hbm.at[1, pl.ds(wid * NBLK, NBLK)], dst_v)

    def zero(i, carry):
        hist_v[pl.ds(i * 16, 16)] = jnp.zeros((16,), jnp.float32)
        return carry

    lax.fori_loop(0, NP // 16, zero, 0)
    ones = jnp.ones((16,), jnp.float32)

    def blk(j, carry):
        for k in range(B // 16):
            idx = dst_v[j, pl.ds(k * 16, 16)]
            plsc.addupdate_scatter(hist_v, [idx], ones)
        return carry

    lax.fori_loop(0, NBLK, blk, 0)
    pltpu.sync_copy(hist_v, out_hbm.at[c, s])


@functools.partial(
    pl.kernel,
    out_type=pltpu.HBM((2, 16, NP), jnp.float32),
    mesh=_MESH,
    scratch_types=[
        pltpu.VMEM((NBLK, B), jnp.int32),
        pltpu.VMEM((NP,), jnp.float32),
    ],
    compiler_params=pltpu.CompilerParams(needs_layout_passes=False),
)
def _deg_kernel(ei_hbm, out_hbm, dst_v, hist_v):
    _deg_body(ei_hbm, out_hbm, dst_v, hist_v)


NCHUNK = 2
CBLK = NBLK // NCHUNK


def _scat_body(xs_hbm, ei_hbm, out_hbm,
               src_v, dst_v, buf0, buf1, acc_sh, sem0, sem1):
    c = lax.axis_index("c")
    s = lax.axis_index("s")
    wid = c * 16 + s

    bufs = [buf0, buf1]
    sems = [sem0, sem1]

    def z(i, carry):
        for k in range(D // 16):
            buf0[i, pl.ds(k * 16, 16)] = jnp.zeros((16,), jnp.float32)
        return carry

    lax.fori_loop(0, B, z, 0)
    base = s * STRIPE
    for t in range(5):
        pltpu.sync_copy(buf0, acc_sh.at[pl.ds(base + t * B, B)])
    plsc.subcore_barrier()

    pltpu.sync_copy(ei_hbm.at[0, pl.ds(wid * NBLK, NBLK)], src_v)
    pltpu.sync_copy(ei_hbm.at[1, pl.ds(wid * NBLK, CBLK)], dst_v)

    for b in range(2):
        pltpu.async_copy(xs_hbm.at[src_v.at[b]], bufs[b], sems[b])

    def make_ring(ci):
        def ring(j, carry):
            for b in range(2):
                i = j * 2 + b
                g = ci * CBLK + i
                pltpu.make_async_copy(xs_hbm.at[src_v.at[g]], bufs[b],
                                      sems[b]).wait()
                pltpu.sync_copy(bufs[b], acc_sh.at[dst_v.at[i]], add=True)
                pltpu.async_copy(xs_hbm.at[src_v.at[(g + 2) % NBLK]],
                                 bufs[b], sems[b])
            return carry
        return ring

    lax.fori_loop(0, CBLK // 2, make_ring(0), 0)
    pltpu.sync_copy(ei_hbm.at[1, pl.ds(wid * NBLK + CBLK, CBLK)], dst_v)
    lax.fori_loop(0, CBLK // 2, make_ring(1), 0)
    for b in range(2):
        pltpu.make_async_copy(xs_hbm.at[src_v.at[b]], bufs[b],
                              sems[b]).wait()

    plsc.subcore_barrier()
    pltpu.sync_copy(acc_sh.at[pl.ds(base, STRIPE)],
                    out_hbm.at[c, pl.ds(base, STRIPE)])


@functools.partial(
    pl.kernel,
    out_type=pltpu.HBM((2, NP, D), jnp.float32),
    mesh=_MESH,
    scratch_types=[
        pltpu.VMEM((NBLK, B), jnp.int32),
        pltpu.VMEM((CBLK, B), jnp.int32),
        pltpu.VMEM((B, D), jnp.float32),
        pltpu.VMEM((B, D), jnp.float32),
        pltpu.VMEM_SHARED((NP, D), jnp.float32),
        pltpu.SemaphoreType.DMA,
        pltpu.SemaphoreType.DMA,
    ],
)
def _scat_kernel(xs_hbm, ei_hbm, out_hbm,
                 src_v, dst_v, buf0, buf1, acc_sh, sem0, sem1):
    _scat_body(xs_hbm, ei_hbm, out_hbm,
               src_v, dst_v, buf0, buf1, acc_sh, sem0, sem1)


def _scale_body(degp_ref, x_ref, xs_ref):
    deg = (jnp.sum(degp_ref[0, :, 0:N], axis=0)
           + jnp.sum(degp_ref[1, :, 0:N], axis=0) + 1.0)
    dinv = lax.rsqrt(deg)[:, None]
    xs_ref[0:N, :] = x_ref[...] * dinv
    xs_ref[N:NP, :] = jnp.zeros((NP - N, D), jnp.float32)


def _scale_call(degp, x):
    return pl.pallas_call(
        _scale_body,
        out_shape=jax.ShapeDtypeStruct((NP, D), jnp.float32),
    )(degp, x)


def _out_body(acc_ref, degp_ref, x_ref, wmu_ref, bmu_ref, wvar_ref, bvar_ref,
              mu_ref, var_ref):
    deg = (jnp.sum(degp_ref[0, :, 0:N], axis=0)
           + jnp.sum(degp_ref[1, :, 0:N], axis=0) + 1.0)
    dinv = lax.rsqrt(deg)[:, None]
    ssum = acc_ref[0, 0:N, :] + acc_ref[1, 0:N, :]
    agg = dinv * (ssum + dinv * x_ref[...])
    mu_ref[...] = jnp.dot(agg, wmu_ref[...],
                          preferred_element_type=jnp.float32) + bmu_ref[...]
    var_ref[...] = jnp.dot(agg, wvar_ref[...],
                           preferred_element_type=jnp.float32) + bvar_ref[...]


def _out_call(acc, degp, x, Wmu, bmu, Wvar, bvar):
    return pl.pallas_call(
        _out_body,
        out_shape=(
            jax.ShapeDtypeStruct((N, D), jnp.float32),
            jax.ShapeDtypeStruct((N, D), jnp.float32),
        ),
    )(acc, degp, x, Wmu, bmu, Wvar, bvar)


def kernel(x, edge_index, Wmu, bmu, Wvar, bvar):
    pad_blk = jnp.asarray(np.stack([
        np.arange(EPAD - E, dtype=np.int32) % N,
        N + np.arange(EPAD - E, dtype=np.int32) % (NP - N),
    ]))
    eip = jnp.concatenate([edge_index, pad_blk], axis=1)
    eip = eip.reshape(2, NBLK_TOT, B)

    degp = _deg_kernel(eip)
    xs = _scale_call(degp, x)
    acc = _scat_kernel(xs, eip)
    mu, var = _out_call(acc, degp, x, Wmu, bmu.reshape(1, D),
                        Wvar, bvar.reshape(1, D))
    return (mu, var)
